# baseline (speedup 1.0000x reference)
import functools

import jax
import jax.numpy as jnp
from jax import lax
from jax.experimental import pallas as pl
from jax.experimental.pallas import tpu as pltpu
from jax.experimental.pallas import tpu_sc as plsc

_VOCAB = 100000
_EMB = 64
_BATCH = 4096
_SEQ = 200
_LANES = 16
_NW = 32
_B_PER_W = _BATCH // _NW
_G0 = 128
_G1 = _SEQ - _G0
_NBUF = 4
_V_PER_W = _VOCAB // _NW
_CHUNK = 125

_FMT = plsc.PackFormat.INTERLEAVED


def _pack_body(table_hbm, out_hbm, in0, in1, po0, po1, si0, si1):
    wid = lax.axis_index("s") * 2 + lax.axis_index("c")
    rbase = wid * _V_PER_W
    ins = (in0, in1)
    pos = (po0, po1)
    sis = (si0, si1)
    n_chunks = _V_PER_W // _CHUNK

    def compute(p, cc):
        pltpu.make_async_copy(
            table_hbm.at[pl.ds(rbase, _CHUNK)], ins[p], sis[p]).wait()

        def row_body(r, carry2):
            v0 = ins[p][r, pl.ds(0, _LANES)]
            v1 = ins[p][r, pl.ds(_LANES, _LANES)]
            v2 = ins[p][r, pl.ds(2 * _LANES, _LANES)]
            v3 = ins[p][r, pl.ds(3 * _LANES, _LANES)]
            pos[p][r, pl.ds(0, 32)] = plsc.pack(v0, v1, format=_FMT)
            pos[p][r, pl.ds(32, 32)] = plsc.pack(v2, v3, format=_FMT)
            return carry2

        lax.fori_loop(0, _CHUNK, row_body, 0)
        pltpu.sync_copy(pos[p], out_hbm.at[pl.ds(rbase + cc * _CHUNK,
                                                 _CHUNK)])

    pltpu.async_copy(table_hbm.at[pl.ds(rbase, _CHUNK)], in0, si0)

    def chunk_body(c, carry):
        for p in range(2):
            cc = c * 2 + p
            pltpu.async_copy(
                table_hbm.at[pl.ds(rbase + (cc + 1) * _CHUNK, _CHUNK)],
                ins[1 - p], sis[1 - p])
            compute(p, cc)
        return carry

    lax.fori_loop(0, (n_chunks - 1) // 2, chunk_body, 0)
    compute(0, jnp.int32(n_chunks - 1))


_pack_call = functools.partial(
    pl.kernel,
    out_type=jax.ShapeDtypeStruct((_VOCAB, _EMB), jnp.bfloat16),
    mesh=plsc.VectorSubcoreMesh(core_axis_name="c", subcore_axis_name="s"),
    compiler_params=pltpu.CompilerParams(use_tc_tiling_on_sc=False,
                                         needs_layout_passes=False),
    scratch_types=[
        pltpu.VMEM((_CHUNK, _EMB), jnp.float32),
        pltpu.VMEM((_CHUNK, _EMB), jnp.float32),
        pltpu.VMEM((_CHUNK, _EMB), jnp.bfloat16),
        pltpu.VMEM((_CHUNK, _EMB), jnp.bfloat16),
        pltpu.SemaphoreType.DMA,
        pltpu.SemaphoreType.DMA,
    ],
)(_pack_body)


def _fire(table_hbm, idx_all, r, buf, sem):
    pltpu.async_copy(table_hbm.at[idx_all.at[r, pl.ds(0, _G0)]],
                     buf.at[pl.ds(0, _G0)], sem)
    pltpu.async_copy(table_hbm.at[idx_all.at[r, pl.ds(_G0, _G1)]],
                     buf.at[pl.ds(_G0, _G1)], sem)


def _drain(table_hbm, idx_all, r, buf, sem):
    pltpu.make_async_copy(table_hbm.at[idx_all.at[r, pl.ds(0, _G0)]],
                          buf.at[pl.ds(0, _G0)], sem).wait()
    pltpu.make_async_copy(table_hbm.at[idx_all.at[r, pl.ds(_G0, _G1)]],
                          buf.at[pl.ds(_G0, _G1)], sem).wait()


def _count_pads(idx_all, r):
    zi = jnp.zeros((_LANES,), jnp.int32)
    oi = jnp.full((_LANES,), 1, jnp.int32)

    def cnt_body(k, acc):
        v = idx_all[r, pl.ds(k * _LANES, _LANES)]
        return acc + jnp.where(v == 0, oi, zi)

    cnt = lax.fori_loop(0, _SEQ // _LANES - 1, cnt_body, zi)
    v11 = idx_all[r, pl.ds(176, _LANES)]
    cnt = cnt + jnp.where(v11 == 0, oi, zi)
    lane = lax.iota(jnp.int32, _LANES)
    vt = idx_all[r, pl.ds(184, _LANES)]
    cnt = cnt + jnp.where((vt == 0) & (lane >= 8), oi, zi)
    n_pad = jnp.int32(0)
    for l in range(_LANES):
        n_pad = n_pad + cnt[l]
    return n_pad


def _consume(buf, n_pad, t0f, out_v, i_out):
    def acc_body(s, accs):
        accs = list(accs)
        for u in range(4):
            r = s * 4 + u
            c = (u % 2) * 4
            for k in range(2):
                fe, fo = plsc.unpack(buf[r, pl.ds(32 * k, 32)], format=_FMT)
                accs[c + 2 * k] = accs[c + 2 * k] + fe
                accs[c + 2 * k + 1] = accs[c + 2 * k + 1] + fo
        return tuple(accs)

    z = jnp.zeros((_LANES,), jnp.float32)
    a = lax.fori_loop(0, _SEQ // 4, acc_body, (z,) * 8)

    npf = jnp.broadcast_to(n_pad.astype(jnp.float32), (_LANES,))
    inv = 1.0 / jnp.maximum(jnp.float32(_SEQ) - npf, 1.0)
    for j in range(4):
        s_j = a[j] + a[4 + j]
        out_v[i_out, pl.ds(j * _LANES, _LANES)] = (
            (s_j - npf * t0f[pl.ds(j * _LANES, _LANES)]) * inv)


def _body(x_hbm, table_hbm, out_hbm,
          idx_all, b0, b1, b2, b3, out_v, t0_b, t0f, s0, s1, s2, s3):
    bufs = (b0, b1, b2, b3)
    sems = (s0, s1, s2, s3)
    wid = lax.axis_index("s") * 2 + lax.axis_index("c")
    base = wid * _B_PER_W

    pltpu.sync_copy(table_hbm.at[0], t0_b)
    for k in range(2):
        fe, fo = plsc.unpack(t0_b[pl.ds(32 * k, 32)], format=_FMT)
        t0f[pl.ds(2 * k * _LANES, _LANES)] = fe
        t0f[pl.ds((2 * k + 1) * _LANES, _LANES)] = fo

    pltpu.sync_copy(x_hbm.at[pl.ds(base, _B_PER_W)], idx_all)

    for b in range(_NBUF - 1):
        _fire(table_hbm, idx_all, jnp.int32(b), bufs[b], sems[b])

    def quad_body(i, carry):
        for b in range(_NBUF):
            r = i * _NBUF + b
            rn = jnp.minimum(r + (_NBUF - 1), _B_PER_W - 1)
            _fire(table_hbm, idx_all, rn, bufs[(b + _NBUF - 1) % _NBUF],
                  sems[(b + _NBUF - 1) % _NBUF])
            n_pad = _count_pads(idx_all, r)
            _drain(table_hbm, idx_all, r, bufs[b], sems[b])
            _consume(bufs[b], n_pad, t0f, out_v, r)
        return carry

    lax.fori_loop(0, _B_PER_W // _NBUF, quad_body, 0)
    last = jnp.int32(_B_PER_W - 1)
    for b in range(_NBUF - 1):
        _drain(table_hbm, idx_all, last, bufs[b], sems[b])

    pltpu.sync_copy(out_v, out_hbm.at[pl.ds(base, _B_PER_W)])


_sc_call = functools.partial(
    pl.kernel,
    out_type=jax.ShapeDtypeStruct((_BATCH, _EMB), jnp.float32),
    mesh=plsc.VectorSubcoreMesh(core_axis_name="c", subcore_axis_name="s"),
    compiler_params=pltpu.CompilerParams(use_tc_tiling_on_sc=False,
                                         needs_layout_passes=False),
    scratch_types=[
        pltpu.VMEM((_B_PER_W, _SEQ), jnp.int32),
        pltpu.VMEM((_SEQ, _EMB), jnp.bfloat16),
        pltpu.VMEM((_SEQ, _EMB), jnp.bfloat16),
        pltpu.VMEM((_SEQ, _EMB), jnp.bfloat16),
        pltpu.VMEM((_SEQ, _EMB), jnp.bfloat16),
        pltpu.VMEM((_B_PER_W, _EMB), jnp.float32),
        pltpu.VMEM((_EMB,), jnp.bfloat16),
        pltpu.VMEM((_EMB,), jnp.float32),
        pltpu.SemaphoreType.DMA,
        pltpu.SemaphoreType.DMA,
        pltpu.SemaphoreType.DMA,
        pltpu.SemaphoreType.DMA,
    ],
)(_body)


def kernel(x, table):
    tp = _pack_call(table)
    return _sc_call(x.astype(jnp.int32), tp)


# speedup vs baseline: 1.0829x; 1.0829x over previous
import functools

import jax
import jax.numpy as jnp
from jax import lax
from jax.experimental import pallas as pl
from jax.experimental.pallas import tpu as pltpu
from jax.experimental.pallas import tpu_sc as plsc

_VOCAB = 100000
_EMB = 64
_BATCH = 4096
_SEQ = 200
_LANES = 16
_NW = 32
_B_PER_W = _BATCH // _NW
_SEGS = ((0, 200),)
_NBUF = 4


def _fire(table_hbm, idx_all, r, buf, sem):
    for off, sz in _SEGS:
        pltpu.async_copy(table_hbm.at[idx_all.at[r, pl.ds(off, sz)]],
                         buf.at[pl.ds(off, sz)], sem)


def _drain(table_hbm, idx_all, r, buf, sem):
    for off, sz in _SEGS:
        pltpu.make_async_copy(table_hbm.at[idx_all.at[r, pl.ds(off, sz)]],
                              buf.at[pl.ds(off, sz)], sem).wait()


def _count_pads(idx_all, r):
    zi = jnp.zeros((_LANES,), jnp.int32)
    oi = jnp.full((_LANES,), 1, jnp.int32)

    def cnt_body(k, acc):
        v = idx_all[r, pl.ds(k * _LANES, _LANES)]
        return acc + jnp.where(v == 0, oi, zi)

    cnt = lax.fori_loop(0, _SEQ // _LANES - 1, cnt_body, zi)
    v11 = idx_all[r, pl.ds(176, _LANES)]
    cnt = cnt + jnp.where(v11 == 0, oi, zi)
    lane = lax.iota(jnp.int32, _LANES)
    vt = idx_all[r, pl.ds(184, _LANES)]
    cnt = cnt + jnp.where((vt == 0) & (lane >= 8), oi, zi)
    n_pad = jnp.int32(0)
    for l in range(_LANES):
        n_pad = n_pad + cnt[l]
    return n_pad


def _consume(buf, n_pad, t0f, out_v, i_out):
    def acc_body(s, accs):
        accs = list(accs)
        for u in range(8):
            r = s * 8 + u
            h = (u % 2) * 4
            for j in range(4):
                accs[h + j] = accs[h + j] + buf[r, pl.ds(j * _LANES, _LANES)]
        return tuple(accs)

    z = jnp.zeros((_LANES,), jnp.float32)
    a = lax.fori_loop(0, _SEQ // 8, acc_body, (z,) * 8)

    npf = jnp.broadcast_to(n_pad.astype(jnp.float32), (_LANES,))
    inv = 1.0 / jnp.maximum(jnp.float32(_SEQ) - npf, 1.0)
    for j in range(4):
        s_j = a[j] + a[4 + j]
        out_v[i_out, pl.ds(j * _LANES, _LANES)] = (
            (s_j - npf * t0f[pl.ds(j * _LANES, _LANES)]) * inv)


def _body(x_hbm, table_hbm, out_hbm,
          idx_all, b0, b1, b2, b3, out_v, t0f, s0, s1, s2, s3):
    bufs = (b0, b1, b2, b3)
    sems = (s0, s1, s2, s3)
    wid = lax.axis_index("s") * 2 + lax.axis_index("c")
    base = wid * _B_PER_W

    pltpu.sync_copy(table_hbm.at[0], t0f)
    pltpu.sync_copy(x_hbm.at[pl.ds(base, _B_PER_W)], idx_all)

    for b in range(_NBUF - 1):
        _fire(table_hbm, idx_all, jnp.int32(b), bufs[b], sems[b])

    def quad_body(i, carry):
        for b in range(_NBUF):
            r = i * _NBUF + b
            rn = jnp.minimum(r + (_NBUF - 1), _B_PER_W - 1)
            _fire(table_hbm, idx_all, rn, bufs[(b + _NBUF - 1) % _NBUF],
                  sems[(b + _NBUF - 1) % _NBUF])
            n_pad = _count_pads(idx_all, r)
            _drain(table_hbm, idx_all, r, bufs[b], sems[b])
            _consume(bufs[b], n_pad, t0f, out_v, r)
        return carry

    lax.fori_loop(0, _B_PER_W // _NBUF, quad_body, 0)
    last = jnp.int32(_B_PER_W - 1)
    for b in range(_NBUF - 1):
        _drain(table_hbm, idx_all, last, bufs[b], sems[b])

    pltpu.sync_copy(out_v, out_hbm.at[pl.ds(base, _B_PER_W)])


_sc_call = functools.partial(
    pl.kernel,
    out_type=jax.ShapeDtypeStruct((_BATCH, _EMB), jnp.float32),
    mesh=plsc.VectorSubcoreMesh(core_axis_name="c", subcore_axis_name="s"),
    compiler_params=pltpu.CompilerParams(use_tc_tiling_on_sc=False,
                                         needs_layout_passes=False),
    scratch_types=[
        pltpu.VMEM((_B_PER_W, _SEQ), jnp.int32),
        pltpu.VMEM((_SEQ, _EMB), jnp.float32),
        pltpu.VMEM((_SEQ, _EMB), jnp.float32),
        pltpu.VMEM((_SEQ, _EMB), jnp.float32),
        pltpu.VMEM((_SEQ, _EMB), jnp.float32),
        pltpu.VMEM((_B_PER_W, _EMB), jnp.float32),
        pltpu.VMEM((_EMB,), jnp.float32),
        pltpu.SemaphoreType.DMA,
        pltpu.SemaphoreType.DMA,
        pltpu.SemaphoreType.DMA,
        pltpu.SemaphoreType.DMA,
    ],
)(_body)


def kernel(x, table):
    return _sc_call(x.astype(jnp.int32), table)
